# baseline (speedup 1.0000x reference)
import jax
import jax.numpy as jnp
from jax.experimental import pallas as pl

_K = 100000
_KP = 100352
_TOPN = 50
_QB = 64


def _dist_body(q_ref, ct_ref, w_ref, b_ref, out_ref):
    q = q_ref[...]
    w = w_ref[...]
    qp = jax.lax.dot_general(
        q, w, (((1,), (1,)), ((), ())), preferred_element_type=jnp.float32
    ) + b_ref[...]
    ct = ct_ref[...]
    csq = jnp.sum(ct * ct, axis=0, keepdims=True)
    qsq = jnp.sum(qp * qp, axis=1, keepdims=True)
    dot = jnp.dot(qp, ct, preferred_element_type=jnp.float32)
    d2 = (qsq + csq) - 2.0 * dot
    out_ref[...] = jnp.sqrt(jnp.maximum(d2, 0.0))


def kernel(query_embeddings, context_embeddings, W, b):
    nq = query_embeddings.shape[0]
    pad = jnp.full((_KP - _K, 16), 1e15, jnp.float32)
    ct = jnp.concatenate([context_embeddings, pad], axis=0).T
    dist = pl.pallas_call(
        _dist_body,
        grid=(nq // _QB,),
        in_specs=[
            pl.BlockSpec((_QB, 16), lambda i: (i, 0)),
            pl.BlockSpec((16, _KP), lambda i: (0, 0)),
            pl.BlockSpec((16, 16), lambda i: (0, 0)),
            pl.BlockSpec((1, 16), lambda i: (0, 0)),
        ],
        out_specs=pl.BlockSpec((_QB, _KP), lambda i: (i, 0)),
        out_shape=jax.ShapeDtypeStruct((nq, _KP), jnp.float32),
    )(query_embeddings, ct, W, b.reshape(1, 16))
    neg, idx = jax.lax.top_k(-dist, _TOPN)
    return (-neg, idx)
